# baseline (speedup 1.0000x reference)
import functools
import jax
import jax.numpy as jnp
import numpy as np
from jax import lax
from jax.experimental import pallas as pl
from jax.experimental.pallas import tpu as pltpu
from jax.experimental.pallas import tpu_sc as plsc

_POOL = 7
_SCALE = 0.0625
_S = 2
_B, _C, _H, _W = 4, 192, 32, 32
_N = 512

_NW = 32
_ROIS_PER_W = _N // _NW
_NBINS = _POOL * _POOL
_BINS_PER_W = _ROIS_PER_W * _NBINS
_K = 16
_UNITS_PER_W = _BINS_PER_W * _K
_CHUNK_BINS = 7
_KP = _K // 2
_CHUNK_ROWS = _CHUNK_BINS * _KP
_N_CHUNKS = _BINS_PER_W // _CHUNK_BINS
_ROI_CHUNKS = _NBINS // _CHUNK_BINS
_CL = _C // 16


def _prep_idx_w(rois, roibatches):
    b = roibatches[:, 0].astype(jnp.int32)
    x1 = rois[:, 0] * _SCALE
    y1 = rois[:, 1] * _SCALE
    x2 = rois[:, 2] * _SCALE
    y2 = rois[:, 3] * _SCALE
    roi_w = jnp.maximum(x2 - x1, 1.0)
    roi_h = jnp.maximum(y2 - y1, 1.0)
    bin_h = roi_h / _POOL
    bin_w = roi_w / _POOL

    u = np.arange(_NBINS * _K)
    kk = u % _K
    binv = u // _K
    phv = jnp.asarray((binv // _POOL).astype(np.float32))
    pwv = jnp.asarray((binv % _POOL).astype(np.float32))
    iyv = jnp.asarray((kk // 8).astype(np.float32))
    ixv = jnp.asarray(((kk // 4) % 2).astype(np.float32))
    cyb = jnp.asarray((kk % 4) // 2 == 1)
    cxb = jnp.asarray((kk % 4) % 2 == 1)

    bh = bin_h[:, None]
    bw = bin_w[:, None]
    y = y1[:, None] + phv[None, :] * bh + (iyv[None, :] + 0.5) * (bh / _S)
    x = x1[:, None] + pwv[None, :] * bw + (ixv[None, :] + 0.5) * (bw / _S)

    def axis_terms(v, size, hib):
        ok = (v >= -1.0) & (v <= size)
        vc = jnp.maximum(v, 0.0)
        v0 = jnp.floor(vc)
        cond = v0 >= (size - 1)
        lo = jnp.where(cond, size - 1, v0)
        hi = jnp.where(cond, size - 1, v0 + 1)
        lw = jnp.where(cond, 0.0, vc - v0)
        r = jnp.where(hib[None, :], hi, lo).astype(jnp.int32)
        wv = jnp.where(hib[None, :], lw, 1.0 - lw)
        return ok, r, wv

    yok, yr, wyv = axis_terms(y, _H, cyb)
    xok, xr, wxv = axis_terms(x, _W, cxb)

    idx = b[:, None] * (_H * _W) + yr * _W + xr
    w = jnp.where(yok & xok, wyv * wxv * (1.0 / (_S * _S)), 0.0)

    idx = idx[:, 0::2].reshape(_NW, _N_CHUNKS, _CHUNK_ROWS).astype(jnp.int32)
    w = w.astype(jnp.float32).reshape(_NW, _UNITS_PER_W)
    return idx, w


@functools.lru_cache(maxsize=None)
def _build_sc_kernel():
    return functools.partial(
        pl.kernel,
        mesh=plsc.VectorSubcoreMesh(core_axis_name="c", subcore_axis_name="s"),
        compiler_params=pltpu.CompilerParams(use_tc_tiling_on_sc=False,
                                             needs_layout_passes=False),
        out_type=jax.ShapeDtypeStruct((_N * _NBINS, _C), jnp.float32),
        scratch_types=[
            pltpu.VMEM((_N_CHUNKS, _CHUNK_ROWS), jnp.int32),
            pltpu.VMEM((_UNITS_PER_W,), jnp.float32),
            pltpu.VMEM((4, _CHUNK_ROWS, _C), jnp.int32),
            pltpu.VMEM((_NBINS, _C), jnp.float32),
            pltpu.SemaphoreType.DMA,
            pltpu.SemaphoreType.DMA,
            pltpu.SemaphoreType.DMA,
            pltpu.SemaphoreType.DMA,
        ],
    )(_roialign_sc_body)


def _roialign_sc_body(idx_hbm, w_hbm, feat_hbm, out_hbm, idx_v, w_v, rows_v,
                      oroi_v, sem_g0, sem_g1, sem_g2, sem_g3):
    wid = lax.axis_index("s") * 2 + lax.axis_index("c")
    pltpu.sync_copy(idx_hbm.at[wid], idx_v)
    pltpu.sync_copy(w_hbm.at[wid], w_v)

    def start(c, buf, sem):
        pltpu.async_copy(feat_hbm.at[idx_v.at[c]], rows_v.at[buf], sem)

    def wait(buf, sem):
        pltpu.make_async_copy(feat_hbm.at[pl.ds(0, _CHUNK_ROWS)],
                              rows_v.at[buf], sem).wait()

    def compute(c, buf):
        def bin_body(bq, carry2):
            u0 = bq * _KP
            wb = w_v[pl.ds((c * _CHUNK_BINS + bq) * _K, _K)]
            accs = [jnp.zeros((16,), jnp.float32) for _ in range(_CL)]
            for k in range(_K):
                wv = jnp.take_along_axis(
                    wb, jnp.full((16,), k, jnp.int32), axis=0,
                    mode="promise_in_bounds")
                for m in range(_CL // 2):
                    x = rows_v[buf, u0 + k // 2,
                               pl.ds((k % 2) * (_C // 2) + m * 16, 16)]
                    a = plsc.bitcast(lax.shift_left(x, 16), jnp.float32)
                    bvals = plsc.bitcast(x, jnp.float32)
                    accs[2 * m] = accs[2 * m] + wv * a
                    accs[2 * m + 1] = accs[2 * m + 1] + wv * bvals
            b = (c % _ROI_CHUNKS) * _CHUNK_BINS + bq
            for j in range(_CL):
                oroi_v[b, pl.ds(j * 16, 16)] = accs[j]
            return carry2

        lax.fori_loop(0, _CHUNK_BINS, bin_body, 0)

    def maybe_flush(c):
        @pl.when(c % _ROI_CHUNKS == _ROI_CHUNKS - 1)
        def _():
            r = c // _ROI_CHUNKS
            pltpu.sync_copy(
                oroi_v,
                out_hbm.at[pl.ds((wid * _ROIS_PER_W + r) * _NBINS, _NBINS)])

    sems = [sem_g0, sem_g1, sem_g2, sem_g3]
    for s in range(3):
        start(s, s, sems[s])

    def quad_body(g, carry):
        c0 = 4 * g
        for s in range(4):
            c = c0 + s

            @pl.when(c + 3 < _N_CHUNKS)
            def _(c=c, s=s):
                start(c + 3, (s + 3) % 4, sems[(s + 3) % 4])

            wait(s, sems[s])
            compute(c, s)
            maybe_flush(c)
        return carry

    lax.fori_loop(0, _N_CHUNKS // 4, quad_body, 0)


def kernel(feat, rois, roibatches):
    featp = jnp.transpose(feat, (0, 2, 3, 1)).reshape(_B * _H * _W, _C)
    featb = (featp.astype(jnp.bfloat16)
             .reshape(_B * _H * _W, _C // 32, 2, 16)
             .transpose(0, 1, 3, 2)
             .reshape(_B * _H * _W, _C // 2, 2))
    feati = lax.bitcast_convert_type(featb, jnp.int32)
    fpad = jnp.concatenate(
        [feati, jnp.zeros((8, _C // 2), jnp.int32)], axis=0)
    fpairs = jnp.concatenate([fpad[:-1], fpad[1:]], axis=1)
    fpairs = jnp.concatenate(
        [fpairs, jnp.zeros((1, _C), jnp.int32)], axis=0)
    idx, w = _prep_idx_w(rois, roibatches)
    out = _build_sc_kernel()(idx, w, fpairs)
    return jnp.transpose(out.reshape(_N, _POOL, _POOL, _C), (0, 3, 1, 2))


# speedup vs baseline: 1.0649x; 1.0649x over previous
import functools
import jax
import jax.numpy as jnp
import numpy as np
from jax import lax
from jax.experimental import pallas as pl
from jax.experimental.pallas import tpu as pltpu
from jax.experimental.pallas import tpu_sc as plsc

_POOL = 7
_SCALE = 0.0625
_S = 2
_B, _C, _H, _W = 4, 192, 32, 32
_N = 512

_NW = 32
_ROIS_PER_W = _N // _NW
_NBINS = _POOL * _POOL
_BINS_PER_W = _ROIS_PER_W * _NBINS
_K = 16
_UNITS_PER_W = _BINS_PER_W * _K
_CHUNK_BINS = 7
_CHUNK_ROWS = _CHUNK_BINS * _K
_N_CHUNKS = _UNITS_PER_W // _CHUNK_ROWS
_ROI_CHUNKS = _NBINS // _CHUNK_BINS
_CL = _C // 16


def _prep_idx_w(rois, roibatches):
    b = roibatches[:, 0].astype(jnp.int32)
    x1 = rois[:, 0] * _SCALE
    y1 = rois[:, 1] * _SCALE
    x2 = rois[:, 2] * _SCALE
    y2 = rois[:, 3] * _SCALE
    roi_w = jnp.maximum(x2 - x1, 1.0)
    roi_h = jnp.maximum(y2 - y1, 1.0)
    bin_h = roi_h / _POOL
    bin_w = roi_w / _POOL

    u = np.arange(_NBINS * _K)
    kk = u % _K
    binv = u // _K
    phv = jnp.asarray((binv // _POOL).astype(np.float32))
    pwv = jnp.asarray((binv % _POOL).astype(np.float32))
    iyv = jnp.asarray((kk // 8).astype(np.float32))
    ixv = jnp.asarray(((kk // 4) % 2).astype(np.float32))
    cyb = jnp.asarray((kk % 4) // 2 == 1)
    cxb = jnp.asarray((kk % 4) % 2 == 1)

    bh = bin_h[:, None]
    bw = bin_w[:, None]
    y = y1[:, None] + phv[None, :] * bh + (iyv[None, :] + 0.5) * (bh / _S)
    x = x1[:, None] + pwv[None, :] * bw + (ixv[None, :] + 0.5) * (bw / _S)

    def axis_terms(v, size, hib):
        ok = (v >= -1.0) & (v <= size)
        vc = jnp.maximum(v, 0.0)
        v0 = jnp.floor(vc)
        cond = v0 >= (size - 1)
        lo = jnp.where(cond, size - 1, v0)
        hi = jnp.where(cond, size - 1, v0 + 1)
        lw = jnp.where(cond, 0.0, vc - v0)
        r = jnp.where(hib[None, :], hi, lo).astype(jnp.int32)
        wv = jnp.where(hib[None, :], lw, 1.0 - lw)
        return ok, r, wv

    yok, yr, wyv = axis_terms(y, _H, cyb)
    xok, xr, wxv = axis_terms(x, _W, cxb)

    idx = b[:, None] * (_H * _W) + yr * _W + xr
    w = jnp.where(yok & xok, wyv * wxv * (1.0 / (_S * _S)), 0.0)

    idx = idx.reshape(_NW, _N_CHUNKS, _CHUNK_ROWS).astype(jnp.int32)
    w = w.astype(jnp.float32).reshape(_NW, _UNITS_PER_W)
    return idx, w


@functools.lru_cache(maxsize=None)
def _build_sc_kernel():
    return functools.partial(
        pl.kernel,
        mesh=plsc.VectorSubcoreMesh(core_axis_name="c", subcore_axis_name="s"),
        compiler_params=pltpu.CompilerParams(use_tc_tiling_on_sc=False,
                                             needs_layout_passes=False),
        out_type=jax.ShapeDtypeStruct((_N * _NBINS, _C), jnp.float32),
        scratch_types=[
            pltpu.VMEM((_N_CHUNKS, _CHUNK_ROWS), jnp.int32),
            pltpu.VMEM((_UNITS_PER_W,), jnp.float32),
            pltpu.VMEM((4, _CHUNK_ROWS, _C // 2), jnp.int32),
            pltpu.VMEM((_NBINS, _C), jnp.float32),
            pltpu.SemaphoreType.DMA,
            pltpu.SemaphoreType.DMA,
            pltpu.SemaphoreType.DMA,
            pltpu.SemaphoreType.DMA,
        ],
    )(_roialign_sc_body)


def _roialign_sc_body(idx_hbm, w_hbm, feat_hbm, out_hbm, idx_v, w_v, rows_v,
                      oroi_v, sem_g0, sem_g1, sem_g2, sem_g3):
    wid = lax.axis_index("s") * 2 + lax.axis_index("c")
    pltpu.sync_copy(idx_hbm.at[wid], idx_v)
    pltpu.sync_copy(w_hbm.at[wid], w_v)

    def start(c, buf, sem):
        pltpu.async_copy(feat_hbm.at[idx_v.at[c]], rows_v.at[buf], sem)

    def wait(buf, sem):
        pltpu.make_async_copy(feat_hbm.at[pl.ds(0, _CHUNK_ROWS)],
                              rows_v.at[buf], sem).wait()

    def compute(c, buf):
        def bin_body(bq, carry2):
            u0 = bq * _K
            wb = w_v[pl.ds(c * _CHUNK_ROWS + u0, _K)]
            accs = [jnp.zeros((16,), jnp.float32) for _ in range(_CL)]
            for k in range(_K):
                wv = jnp.take_along_axis(
                    wb, jnp.full((16,), k, jnp.int32), axis=0,
                    mode="promise_in_bounds")
                for m in range(_CL // 2):
                    x = rows_v[buf, u0 + k, pl.ds(m * 16, 16)]
                    a = plsc.bitcast(lax.shift_left(x, 16), jnp.float32)
                    bvals = plsc.bitcast(x, jnp.float32)
                    accs[2 * m] = accs[2 * m] + wv * a
                    accs[2 * m + 1] = accs[2 * m + 1] + wv * bvals
            b = (c % _ROI_CHUNKS) * _CHUNK_BINS + bq
            for j in range(_CL):
                oroi_v[b, pl.ds(j * 16, 16)] = accs[j]
            return carry2

        lax.fori_loop(0, _CHUNK_BINS, bin_body, 0)

    def maybe_flush(c):
        @pl.when(c % _ROI_CHUNKS == _ROI_CHUNKS - 1)
        def _():
            r = c // _ROI_CHUNKS
            pltpu.sync_copy(
                oroi_v,
                out_hbm.at[pl.ds((wid * _ROIS_PER_W + r) * _NBINS, _NBINS)])

    sems = [sem_g0, sem_g1, sem_g2, sem_g3]
    for s in range(3):
        start(s, s, sems[s])

    def quad_body(g, carry):
        c0 = 4 * g
        for s in range(4):
            c = c0 + s

            @pl.when(c + 3 < _N_CHUNKS)
            def _(c=c, s=s):
                start(c + 3, (s + 3) % 4, sems[(s + 3) % 4])

            wait(s, sems[s])
            compute(c, s)
            maybe_flush(c)
        return carry

    lax.fori_loop(0, _N_CHUNKS // 4, quad_body, 0)


def kernel(feat, rois, roibatches):
    featp = jnp.transpose(feat, (0, 2, 3, 1)).reshape(_B * _H * _W, _C)
    featb = (featp.astype(jnp.bfloat16)
             .reshape(_B * _H * _W, _C // 32, 2, 16)
             .transpose(0, 1, 3, 2)
             .reshape(_B * _H * _W, _C // 2, 2))
    feati = lax.bitcast_convert_type(featb, jnp.int32)
    idx, w = _prep_idx_w(rois, roibatches)
    out = _build_sc_kernel()(idx, w, feati)
    return jnp.transpose(out.reshape(_N, _POOL, _POOL, _C), (0, 3, 1, 2))


# speedup vs baseline: 1.1242x; 1.0557x over previous
import functools
import jax
import jax.numpy as jnp
import numpy as np
from jax import lax
from jax.experimental import pallas as pl
from jax.experimental.pallas import tpu as pltpu
from jax.experimental.pallas import tpu_sc as plsc

_POOL = 7
_SCALE = 0.0625
_S = 2
_B, _C, _H, _W = 4, 192, 32, 32
_N = 512

_NW = 32
_ROIS_PER_W = _N // _NW
_NBINS = _POOL * _POOL
_BINS_PER_W = _ROIS_PER_W * _NBINS
_K = 16
_UNITS_PER_W = _BINS_PER_W * _K
_CHUNK_BINS = 7
_CHUNK_ROWS = _CHUNK_BINS * _K
_N_CHUNKS = _UNITS_PER_W // _CHUNK_ROWS
_ROI_CHUNKS = _NBINS // _CHUNK_BINS
_CL = _C // 16


def _prep_idx_w(rois, roibatches):
    b = roibatches[:, 0].astype(jnp.int32)
    x1 = rois[:, 0] * _SCALE
    y1 = rois[:, 1] * _SCALE
    x2 = rois[:, 2] * _SCALE
    y2 = rois[:, 3] * _SCALE
    roi_w = jnp.maximum(x2 - x1, 1.0)
    roi_h = jnp.maximum(y2 - y1, 1.0)
    bin_h = roi_h / _POOL
    bin_w = roi_w / _POOL

    u = np.arange(_NBINS * _K)
    kk = u % _K
    binv = u // _K
    phv = jnp.asarray((binv // _POOL).astype(np.float32))
    pwv = jnp.asarray((binv % _POOL).astype(np.float32))
    iyv = jnp.asarray((kk // 8).astype(np.float32))
    ixv = jnp.asarray(((kk // 4) % 2).astype(np.float32))
    cyb = jnp.asarray((kk % 4) // 2 == 1)
    cxb = jnp.asarray((kk % 4) % 2 == 1)

    bh = bin_h[:, None]
    bw = bin_w[:, None]
    y = y1[:, None] + phv[None, :] * bh + (iyv[None, :] + 0.5) * (bh / _S)
    x = x1[:, None] + pwv[None, :] * bw + (ixv[None, :] + 0.5) * (bw / _S)

    def axis_terms(v, size, hib):
        ok = (v >= -1.0) & (v <= size)
        vc = jnp.maximum(v, 0.0)
        v0 = jnp.floor(vc)
        cond = v0 >= (size - 1)
        lo = jnp.where(cond, size - 1, v0)
        hi = jnp.where(cond, size - 1, v0 + 1)
        lw = jnp.where(cond, 0.0, vc - v0)
        r = jnp.where(hib[None, :], hi, lo).astype(jnp.int32)
        wv = jnp.where(hib[None, :], lw, 1.0 - lw)
        return ok, r, wv

    yok, yr, wyv = axis_terms(y, _H, cyb)
    xok, xr, wxv = axis_terms(x, _W, cxb)

    idx = b[:, None] * (_H * _W) + yr * _W + xr
    w = jnp.where(yok & xok, wyv * wxv * (1.0 / (_S * _S)), 0.0)

    return idx.astype(jnp.int32), w.astype(jnp.float32)


@functools.lru_cache(maxsize=None)
def _build_sc_kernel():
    return functools.partial(
        pl.kernel,
        mesh=plsc.VectorSubcoreMesh(core_axis_name="c", subcore_axis_name="s"),
        compiler_params=pltpu.CompilerParams(use_tc_tiling_on_sc=False,
                                             needs_layout_passes=False),
        out_type=jax.ShapeDtypeStruct((_N * _NBINS, _C), jnp.float32),
        scratch_types=[
            pltpu.VMEM((_ROIS_PER_W, _NBINS * _K), jnp.int32),
            pltpu.VMEM((_ROIS_PER_W, _NBINS * _K), jnp.float32),
            pltpu.VMEM((6, _CHUNK_ROWS, _C // 2), jnp.int32),
            pltpu.VMEM((_NBINS, _C), jnp.float32),
        ] + [pltpu.SemaphoreType.DMA] * 6,
    )(_roialign_sc_body)


def _roialign_sc_body(idx_hbm, w_hbm, feat_hbm, out_hbm, idx_v, w_v, rows_v,
                      oroi_v, sem_g0, sem_g1, sem_g2, sem_g3, sem_g4, sem_g5):
    wid = lax.axis_index("s") * 2 + lax.axis_index("c")
    pltpu.sync_copy(idx_hbm.at[pl.ds(wid * _ROIS_PER_W, _ROIS_PER_W)], idx_v)
    pltpu.sync_copy(w_hbm.at[pl.ds(wid * _ROIS_PER_W, _ROIS_PER_W)], w_v)

    def start(c, buf, sem):
        ichunk = idx_v.at[c // _ROI_CHUNKS,
                          pl.ds((c % _ROI_CHUNKS) * _CHUNK_ROWS, _CHUNK_ROWS)]
        pltpu.async_copy(feat_hbm.at[ichunk], rows_v.at[buf], sem)

    def wait(buf, sem):
        pltpu.make_async_copy(feat_hbm.at[pl.ds(0, _CHUNK_ROWS)],
                              rows_v.at[buf], sem).wait()

    def compute(c, buf):
        def bin_body(bq, carry2):
            u0 = bq * _K
            wb = w_v[c // _ROI_CHUNKS,
                     pl.ds((c % _ROI_CHUNKS) * _CHUNK_ROWS + u0, _K)]
            accs = [jnp.zeros((16,), jnp.float32) for _ in range(_CL)]
            for k in range(_K):
                wv = jnp.take_along_axis(
                    wb, jnp.full((16,), k, jnp.int32), axis=0,
                    mode="promise_in_bounds")
                for m in range(_CL // 2):
                    x = rows_v[buf, u0 + k, pl.ds(m * 16, 16)]
                    a = plsc.bitcast(lax.shift_left(x, 16), jnp.float32)
                    bvals = plsc.bitcast(x, jnp.float32)
                    accs[2 * m] = accs[2 * m] + wv * a
                    accs[2 * m + 1] = accs[2 * m + 1] + wv * bvals
            b = (c % _ROI_CHUNKS) * _CHUNK_BINS + bq
            for j in range(_CL):
                oroi_v[b, pl.ds(j * 16, 16)] = accs[j]
            return carry2

        lax.fori_loop(0, _CHUNK_BINS, bin_body, 0)

    def maybe_flush(c):
        @pl.when(c % _ROI_CHUNKS == _ROI_CHUNKS - 1)
        def _():
            r = c // _ROI_CHUNKS
            pltpu.sync_copy(
                oroi_v,
                out_hbm.at[pl.ds((wid * _ROIS_PER_W + r) * _NBINS, _NBINS)])

    sems = [sem_g0, sem_g1, sem_g2, sem_g3, sem_g4, sem_g5]
    for s in range(5):
        start(s, s, sems[s])

    def hex_body(g, carry):
        c0 = 6 * g
        for s in range(6):
            c = c0 + s

            @pl.when(c + 5 < _N_CHUNKS)
            def _(c=c, s=s):
                start(c + 5, (s + 5) % 6, sems[(s + 5) % 6])

            @pl.when(c < _N_CHUNKS)
            def _(c=c, s=s):
                wait(s, sems[s])
                compute(c, s)

            maybe_flush(c)
        return carry

    lax.fori_loop(0, (_N_CHUNKS + 5) // 6, hex_body, 0)


def kernel(feat, rois, roibatches):
    featb = jnp.transpose(
        feat.astype(jnp.bfloat16).reshape(_B, _C // 32, 2, 16, _H, _W),
        (0, 4, 5, 1, 3, 2))
    feati = lax.bitcast_convert_type(
        featb.reshape(_B * _H * _W, _C // 2, 2), jnp.int32)
    idx, w = _prep_idx_w(rois, roibatches)
    out = _build_sc_kernel()(idx, w, feati)
    return jnp.transpose(out.reshape(_N, _POOL, _POOL, _C), (0, 3, 1, 2))


# speedup vs baseline: 1.3268x; 1.1802x over previous
import functools
import jax
import jax.numpy as jnp
import numpy as np
from jax import lax
from jax.experimental import pallas as pl
from jax.experimental.pallas import tpu as pltpu
from jax.experimental.pallas import tpu_sc as plsc

_POOL = 7
_SCALE = 0.0625
_S = 2
_B, _C, _H, _W = 4, 192, 32, 32
_N = 512

_NW = 32
_ROIS_PER_W = _N // _NW
_NBINS = _POOL * _POOL
_BINS_PER_W = _ROIS_PER_W * _NBINS
_K = 16
_UNITS_PER_W = _BINS_PER_W * _K
_CHUNK_BINS = 7
_CHUNK_ROWS = _CHUNK_BINS * _K
_N_CHUNKS = _UNITS_PER_W // _CHUNK_ROWS
_ROI_CHUNKS = _NBINS // _CHUNK_BINS
_CL = _C // 16


def _prep_idx_w(rois, roibatches):
    b = roibatches[:, 0].astype(jnp.int32)
    x1 = rois[:, 0] * _SCALE
    y1 = rois[:, 1] * _SCALE
    x2 = rois[:, 2] * _SCALE
    y2 = rois[:, 3] * _SCALE
    roi_w = jnp.maximum(x2 - x1, 1.0)
    roi_h = jnp.maximum(y2 - y1, 1.0)
    bin_h = roi_h / _POOL
    bin_w = roi_w / _POOL

    u = np.arange(_NBINS * _K)
    kk = u % _K
    binv = u // _K
    phv = jnp.asarray((binv // _POOL).astype(np.float32))
    pwv = jnp.asarray((binv % _POOL).astype(np.float32))
    iyv = jnp.asarray((kk // 8).astype(np.float32))
    ixv = jnp.asarray(((kk // 4) % 2).astype(np.float32))
    cyb = jnp.asarray((kk % 4) // 2 == 1)
    cxb = jnp.asarray((kk % 4) % 2 == 1)

    bh = bin_h[:, None]
    bw = bin_w[:, None]
    y = y1[:, None] + phv[None, :] * bh + (iyv[None, :] + 0.5) * (bh / _S)
    x = x1[:, None] + pwv[None, :] * bw + (ixv[None, :] + 0.5) * (bw / _S)

    def axis_terms(v, size, hib):
        ok = (v >= -1.0) & (v <= size)
        vc = jnp.maximum(v, 0.0)
        v0 = jnp.floor(vc)
        cond = v0 >= (size - 1)
        lo = jnp.where(cond, size - 1, v0)
        hi = jnp.where(cond, size - 1, v0 + 1)
        lw = jnp.where(cond, 0.0, vc - v0)
        r = jnp.where(hib[None, :], hi, lo).astype(jnp.int32)
        wv = jnp.where(hib[None, :], lw, 1.0 - lw)
        return ok, r, wv

    yok, yr, wyv = axis_terms(y, _H, cyb)
    xok, xr, wxv = axis_terms(x, _W, cxb)

    idx = b[:, None] * (_H * _W) + yr * _W + xr
    w = jnp.where(yok & xok, wyv * wxv * (1.0 / (_S * _S)), 0.0)

    return idx.astype(jnp.int32), w.astype(jnp.float32)


@functools.lru_cache(maxsize=None)
def _build_sc_kernel():
    return functools.partial(
        pl.kernel,
        mesh=plsc.VectorSubcoreMesh(core_axis_name="c", subcore_axis_name="s"),
        compiler_params=pltpu.CompilerParams(use_tc_tiling_on_sc=False,
                                             needs_layout_passes=False),
        out_type=jax.ShapeDtypeStruct((_N * _NBINS, _C), jnp.float32),
        scratch_types=[
            pltpu.VMEM((_ROIS_PER_W, _NBINS * _K), jnp.int32),
            pltpu.VMEM((_ROIS_PER_W, _NBINS * _K), jnp.float32),
            pltpu.VMEM((6, _CHUNK_ROWS, _C // 2), jnp.int32),
            pltpu.VMEM((_NBINS, _C), jnp.float32),
            pltpu.VMEM_SHARED((_B * _H * _W, _C // 2), jnp.int32),
        ] + [pltpu.SemaphoreType.DMA] * 6,
    )(_roialign_sc_body)


def _roialign_sc_body(idx_hbm, w_hbm, feat_hbm, out_hbm, idx_v, w_v, rows_v,
                      oroi_v, feat_sh, sem_g0, sem_g1, sem_g2, sem_g3, sem_g4,
                      sem_g5):
    wid = lax.axis_index("s") * 2 + lax.axis_index("c")

    @pl.when(lax.axis_index("s") == 0)
    def _():
        pltpu.sync_copy(feat_hbm, feat_sh)

    pltpu.sync_copy(idx_hbm.at[pl.ds(wid * _ROIS_PER_W, _ROIS_PER_W)], idx_v)
    pltpu.sync_copy(w_hbm.at[pl.ds(wid * _ROIS_PER_W, _ROIS_PER_W)], w_v)
    plsc.subcore_barrier()

    def start(c, buf, sem):
        ichunk = idx_v.at[c // _ROI_CHUNKS,
                          pl.ds((c % _ROI_CHUNKS) * _CHUNK_ROWS, _CHUNK_ROWS)]
        pltpu.async_copy(feat_sh.at[ichunk], rows_v.at[buf], sem)

    def wait(buf, sem):
        pltpu.make_async_copy(feat_hbm.at[pl.ds(0, _CHUNK_ROWS)],
                              rows_v.at[buf], sem).wait()

    def compute(c, buf):
        def bin_body(bq, carry2):
            u0 = bq * _K
            wb = w_v[c // _ROI_CHUNKS,
                     pl.ds((c % _ROI_CHUNKS) * _CHUNK_ROWS + u0, _K)]
            accs = [jnp.zeros((16,), jnp.float32) for _ in range(_CL)]
            for k in range(_K):
                wv = jnp.take_along_axis(
                    wb, jnp.full((16,), k, jnp.int32), axis=0,
                    mode="promise_in_bounds")
                for m in range(_CL // 2):
                    x = rows_v[buf, u0 + k, pl.ds(m * 16, 16)]
                    a = plsc.bitcast(lax.shift_left(x, 16), jnp.float32)
                    bvals = plsc.bitcast(x, jnp.float32)
                    accs[2 * m] = accs[2 * m] + wv * a
                    accs[2 * m + 1] = accs[2 * m + 1] + wv * bvals
            b = (c % _ROI_CHUNKS) * _CHUNK_BINS + bq
            for j in range(_CL):
                oroi_v[b, pl.ds(j * 16, 16)] = accs[j]
            return carry2

        lax.fori_loop(0, _CHUNK_BINS, bin_body, 0)

    def maybe_flush(c):
        @pl.when(c % _ROI_CHUNKS == _ROI_CHUNKS - 1)
        def _():
            r = c // _ROI_CHUNKS
            pltpu.sync_copy(
                oroi_v,
                out_hbm.at[pl.ds((wid * _ROIS_PER_W + r) * _NBINS, _NBINS)])

    sems = [sem_g0, sem_g1, sem_g2, sem_g3, sem_g4, sem_g5]
    for s in range(5):
        start(s, s, sems[s])

    def hex_body(g, carry):
        c0 = 6 * g
        for s in range(6):
            c = c0 + s

            @pl.when(c + 5 < _N_CHUNKS)
            def _(c=c, s=s):
                start(c + 5, (s + 5) % 6, sems[(s + 5) % 6])

            @pl.when(c < _N_CHUNKS)
            def _(c=c, s=s):
                wait(s, sems[s])
                compute(c, s)

            maybe_flush(c)
        return carry

    lax.fori_loop(0, (_N_CHUNKS + 5) // 6, hex_body, 0)


def kernel(feat, rois, roibatches):
    featb = jnp.transpose(
        feat.astype(jnp.bfloat16).reshape(_B, _C // 32, 2, 16, _H, _W),
        (0, 4, 5, 1, 3, 2))
    feati = lax.bitcast_convert_type(
        featb.reshape(_B * _H * _W, _C // 2, 2), jnp.int32)
    idx, w = _prep_idx_w(rois, roibatches)
    out = _build_sc_kernel()(idx, w, feati)
    return jnp.transpose(out.reshape(_N, _POOL, _POOL, _C), (0, 3, 1, 2))
